# baseline (speedup 1.0000x reference)
import functools

import jax
import jax.numpy as jnp
from jax import lax
from jax.experimental import pallas as pl
from jax.experimental.pallas import tpu as pltpu
from jax.experimental.pallas import tpu_sc as plsc

N_NODES = 10000
D = 128
N_EDGES = 320000

NC = 2
NS = 16
NW = NC * NS
COLS = D // NC
CHUNK = 128
NBUF = 4
E_PAD = 327680
NCHUNK_DEG = E_PAD // (NW * CHUNK)
NCHUNK = E_PAD // (NS * CHUNK)
N_PAD = 10240
ROWS_PER_TILE = N_PAD // NS

_mesh = plsc.VectorSubcoreMesh(core_axis_name="c", subcore_axis_name="s")


@functools.partial(
    pl.kernel,
    out_type=jax.ShapeDtypeStruct((NC, N_PAD // 16, 16), jnp.float32),
    mesh=_mesh,
    scratch_types=[
        pltpu.VMEM((NCHUNK_DEG, CHUNK), jnp.int32),
        pltpu.VMEM((N_PAD,), jnp.float32),
        pltpu.VMEM((N_PAD // 16, 16), jnp.float32),
        pltpu.VMEM((5, 128), jnp.int32),
        pltpu.VMEM_SHARED((N_PAD // 16, 16), jnp.float32),
    ],
    compiler_params=pltpu.CompilerParams(needs_layout_passes=False,
                                         use_tc_tiling_on_sc=False),
)
def _deg_kernel(dst_hbm, rowid_hbm, out_hbm, dst_v, deg_v, deg2_v, rowid_v,
                deg_sh):
    cid = lax.axis_index("c")
    sid = lax.axis_index("s")
    wid = cid * NS + sid
    pltpu.sync_copy(dst_hbm.at[wid], dst_v)
    pltpu.sync_copy(rowid_hbm, rowid_v)

    zeros16 = jnp.zeros((16,), jnp.float32)

    def _zero(r, carry):
        deg_v[pl.ds(r * 16, 16)] = zeros16
        deg2_v[r, :] = zeros16
        return carry

    lax.fori_loop(0, N_PAD // 16, _zero, 0)

    @pl.when(sid == 0)
    def _():
        pltpu.sync_copy(deg2_v, deg_sh)

    plsc.subcore_barrier()

    ones16 = jnp.ones((16,), jnp.float32)

    def _edges(j, carry):
        def _sub(k, c2):
            idx = dst_v[j, pl.ds(k * 16, 16)]
            plsc.addupdate_scatter(deg_v, [idx], ones16)
            return c2
        return lax.fori_loop(0, CHUNK // 16, _sub, carry)

    lax.fori_loop(0, NCHUNK_DEG, _edges, 0)

    def _tr(r, carry):
        deg2_v[r, :] = deg_v[pl.ds(r * 16, 16)]
        return carry

    lax.fori_loop(0, N_PAD // 16, _tr, 0)

    def _comb(c, carry):
        pltpu.sync_copy(deg2_v.at[pl.ds(c * 128, 128)],
                        deg_sh.at[rowid_v.at[c]], add=True)
        return carry

    lax.fori_loop(0, (N_PAD // 16) // 128, _comb, 0)
    plsc.subcore_barrier()

    @pl.when(sid == 0)
    def _():
        pltpu.sync_copy(deg_sh, out_hbm.at[cid])


@functools.partial(
    pl.kernel,
    out_type=jax.ShapeDtypeStruct((NC, N_PAD, COLS), jnp.float32),
    mesh=_mesh,
    scratch_types=[
        pltpu.VMEM((NCHUNK, CHUNK), jnp.int32),
        pltpu.VMEM((NCHUNK, CHUNK), jnp.int32),
        pltpu.VMEM((CHUNK, COLS), jnp.float32),
        pltpu.VMEM((CHUNK, COLS), jnp.float32),
        pltpu.VMEM((CHUNK, COLS), jnp.float32),
        pltpu.VMEM((CHUNK, COLS), jnp.float32),
        pltpu.VMEM_SHARED((N_PAD, COLS), jnp.float32),
        pltpu.SemaphoreType.DMA,
        pltpu.SemaphoreType.DMA,
        pltpu.SemaphoreType.DMA,
        pltpu.SemaphoreType.DMA,
    ],
    compiler_params=pltpu.CompilerParams(needs_layout_passes=False,
                                         use_tc_tiling_on_sc=False),
)
def _scatter_kernel(u_hbm, src_hbm, dst_hbm, out_hbm,
                    src_v, dst_v, r0b, r1b, r2b, r3b, acc, s0, s1, s2, s3):
    rows = (r0b, r1b, r2b, r3b)
    sems = (s0, s1, s2, s3)
    cid = lax.axis_index("c")
    sid = lax.axis_index("s")
    u_half = u_hbm.at[pl.ds(cid * N_PAD, N_PAD)]
    pltpu.sync_copy(src_hbm.at[sid], src_v)
    pltpu.sync_copy(dst_hbm.at[sid], dst_v)
    r0 = sid * ROWS_PER_TILE
    pltpu.sync_copy(u_hbm.at[pl.ds(cid * N_PAD + r0, ROWS_PER_TILE)],
                    acc.at[pl.ds(r0, ROWS_PER_TILE)])
    plsc.subcore_barrier()

    for b in range(NBUF):
        pltpu.async_copy(u_half.at[src_v.at[b]], rows[b], sems[b])

    def _group(g, carry):
        for b in range(NBUF):
            j = g * NBUF + b
            pltpu.make_async_copy(u_half.at[src_v.at[j]], rows[b],
                                  sems[b]).wait()
            pltpu.sync_copy(rows[b], acc.at[dst_v.at[j]], add=True)
            pltpu.async_copy(u_half.at[src_v.at[j + NBUF]], rows[b], sems[b])
        return carry

    lax.fori_loop(0, NCHUNK // NBUF - 1, _group, 0)
    for b in range(NBUF):
        j = NCHUNK - NBUF + b
        pltpu.make_async_copy(u_half.at[src_v.at[j]], rows[b], sems[b]).wait()
        pltpu.sync_copy(rows[b], acc.at[dst_v.at[j]], add=True)

    plsc.subcore_barrier()
    pltpu.sync_copy(acc.at[pl.ds(r0, ROWS_PER_TILE)],
                    out_hbm.at[cid, pl.ds(r0, ROWS_PER_TILE)])


_BR = 256
_GRID = N_PAD // _BR


def _dinv(d0, d1):
    return lax.rsqrt(d0 + d1 + 1.0)


def _u_body(x_ref, w_ref, d0_ref, d1_ref, u_ref):
    h = jnp.dot(x_ref[...], w_ref[...], preferred_element_type=jnp.float32)
    u = h * _dinv(d0_ref[...], d1_ref[...])
    u_ref[0] = u[:, :COLS]
    u_ref[1] = u[:, COLS:]


def _ln_relu(pre, g, beta):
    mu = jnp.mean(pre, axis=1, keepdims=True)
    var = jnp.mean((pre - mu) ** 2, axis=1, keepdims=True)
    return jnp.maximum((pre - mu) * lax.rsqrt(var + 1e-5) * g + beta, 0.0)


def _mid_body(p_ref, x0_ref, w2_ref, b1_ref, g1_ref,
              be1_ref, d0_ref, d1_ref, x1_ref, u2_ref):
    dinv = _dinv(d0_ref[...], d1_ref[...])
    scat = jnp.concatenate([p_ref[0], p_ref[1]], axis=1)
    pre = scat * dinv + b1_ref[...]
    x1 = _ln_relu(pre, g1_ref[...], be1_ref[...]) + x0_ref[...]
    x1_ref[...] = x1
    u2 = jnp.dot(x1, w2_ref[...], preferred_element_type=jnp.float32) * dinv
    u2_ref[0] = u2[:, :COLS]
    u2_ref[1] = u2[:, COLS:]


def _final_body(p_ref, x1_ref, b2_ref, g2_ref, be2_ref,
                d0_ref, d1_ref, o_ref):
    dinv = _dinv(d0_ref[...], d1_ref[...])
    scat = jnp.concatenate([p_ref[0], p_ref[1]], axis=1)
    pre = scat * dinv + b2_ref[...]
    o_ref[...] = _ln_relu(pre, g2_ref[...], be2_ref[...]) + x1_ref[...]


def _row_spec():
    return pl.BlockSpec((_BR, D), lambda i: (i, 0))


def _stk_spec():
    return pl.BlockSpec((NC, _BR, COLS), lambda i: (0, i, 0))


def _full_spec():
    return pl.BlockSpec((D, D), lambda i: (0, 0))


def _vec_spec():
    return pl.BlockSpec((1, D), lambda i: (0, 0))


def _col_spec():
    return pl.BlockSpec((_BR, 1), lambda i: (i, 0))


_f32 = jnp.float32


def _u_call(xp, W, d0, d1):
    return pl.pallas_call(
        _u_body,
        grid=(_GRID,),
        in_specs=[_row_spec(), _full_spec(), _col_spec(), _col_spec()],
        out_specs=_stk_spec(),
        out_shape=jax.ShapeDtypeStruct((NC, N_PAD, COLS), _f32),
    )(xp, W, d0, d1)


def _mid_call(parts, x0, W2, b1, g1, be1, d0, d1):
    return pl.pallas_call(
        _mid_body,
        grid=(_GRID,),
        in_specs=[_stk_spec(), _row_spec(),
                  _full_spec(), _vec_spec(), _vec_spec(), _vec_spec(),
                  _col_spec(), _col_spec()],
        out_specs=[_row_spec(), _stk_spec()],
        out_shape=[jax.ShapeDtypeStruct((N_PAD, D), _f32),
                   jax.ShapeDtypeStruct((NC, N_PAD, COLS), _f32)],
    )(parts, x0, W2, b1, g1, be1, d0, d1)


def _final_call(parts, x1, b2, g2, be2, d0, d1):
    return pl.pallas_call(
        _final_body,
        grid=(_GRID,),
        in_specs=[_stk_spec(), _row_spec(),
                  _vec_spec(), _vec_spec(), _vec_spec(), _col_spec(),
                  _col_spec()],
        out_specs=_row_spec(),
        out_shape=jax.ShapeDtypeStruct((N_PAD, D), _f32),
    )(parts, x1, b2, g2, be2, d0, d1)


def kernel(x, edge_index, W1, b1, g1, beta1, W2, b2, g2, beta2):
    ei = edge_index.astype(jnp.int32)
    pad = E_PAD - N_EDGES
    dummy = N_NODES + jnp.arange(pad, dtype=jnp.int32) % (N_PAD - N_NODES)
    src = jnp.concatenate([ei[0], dummy])
    dst = jnp.concatenate([ei[1], dummy])
    src3 = src.reshape(NS, NCHUNK, CHUNK)
    dst3 = dst.reshape(NS, NCHUNK, CHUNK)
    dst_deg = dst.reshape(NW, NCHUNK_DEG, CHUNK)
    rowid = jnp.arange(N_PAD // 16, dtype=jnp.int32).reshape(5, 128)
    xp = jnp.pad(x, ((0, N_PAD - N_NODES), (0, 0)))

    degp = _deg_kernel(dst_deg, rowid)
    d0 = degp[0].reshape(N_PAD, 1)
    d1 = degp[1].reshape(N_PAD, 1)

    b1r = b1.reshape(1, D)
    g1r = g1.reshape(1, D)
    be1r = beta1.reshape(1, D)
    b2r = b2.reshape(1, D)
    g2r = g2.reshape(1, D)
    be2r = beta2.reshape(1, D)

    u1 = _u_call(xp, W1, d0, d1)
    parts1 = _scatter_kernel(u1.reshape(NC * N_PAD, COLS), src3, dst3)
    x1, u2 = _mid_call(parts1, xp, W2, b1r, g1r, be1r, d0, d1)
    parts2 = _scatter_kernel(u2.reshape(NC * N_PAD, COLS), src3, dst3)
    x2 = _final_call(parts2, x1, b2r, g2r, be2r, d0, d1)
    return x2[:N_NODES]


# speedup vs baseline: 1.0302x; 1.0302x over previous
import functools

import jax
import jax.numpy as jnp
from jax import lax
from jax.experimental import pallas as pl
from jax.experimental.pallas import tpu as pltpu
from jax.experimental.pallas import tpu_sc as plsc

N_NODES = 10000
D = 128
N_EDGES = 320000

NC = 2
NS = 16
NW = NC * NS
COLS = D // NC
CHUNK = 128
NBUF = 4
E_PAD = 327680
NCHUNK_DEG = E_PAD // (NW * CHUNK)
NCHUNK = E_PAD // (NS * CHUNK)
N_PAD = 10240
ROWS_PER_TILE = N_PAD // NS

_mesh = plsc.VectorSubcoreMesh(core_axis_name="c", subcore_axis_name="s")


@functools.partial(
    pl.kernel,
    out_type=jax.ShapeDtypeStruct((NW, N_PAD), jnp.float32),
    mesh=_mesh,
    scratch_types=[
        pltpu.VMEM((NCHUNK_DEG, CHUNK), jnp.int32),
        pltpu.VMEM((N_PAD,), jnp.float32),
    ],
    compiler_params=pltpu.CompilerParams(needs_layout_passes=False),
)
def _deg_kernel(dst_hbm, out_hbm, dst_v, deg_v):
    cid = lax.axis_index("c")
    sid = lax.axis_index("s")
    wid = cid * NS + sid
    pltpu.sync_copy(dst_hbm.at[wid], dst_v)

    zeros16 = jnp.zeros((16,), jnp.float32)

    def _zero(r, carry):
        deg_v[pl.ds(r * 16, 16)] = zeros16
        return carry

    lax.fori_loop(0, N_PAD // 16, _zero, 0)

    ones16 = jnp.ones((16,), jnp.float32)

    def _edges(j, carry):
        def _sub(k, c2):
            idx = dst_v[j, pl.ds(k * 16, 16)]
            plsc.addupdate_scatter(deg_v, [idx], ones16)
            return c2
        return lax.fori_loop(0, CHUNK // 16, _sub, carry)

    lax.fori_loop(0, NCHUNK_DEG, _edges, 0)
    pltpu.sync_copy(deg_v, out_hbm.at[wid])


@functools.partial(
    pl.kernel,
    out_type=jax.ShapeDtypeStruct((NC, N_PAD, COLS), jnp.float32),
    mesh=_mesh,
    scratch_types=[
        pltpu.VMEM((NCHUNK, CHUNK), jnp.int32),
        pltpu.VMEM((NCHUNK, CHUNK), jnp.int32),
        pltpu.VMEM((CHUNK, COLS), jnp.float32),
        pltpu.VMEM((CHUNK, COLS), jnp.float32),
        pltpu.VMEM((CHUNK, COLS), jnp.float32),
        pltpu.VMEM((CHUNK, COLS), jnp.float32),
        pltpu.VMEM_SHARED((N_PAD, COLS), jnp.float32),
        pltpu.SemaphoreType.DMA,
        pltpu.SemaphoreType.DMA,
        pltpu.SemaphoreType.DMA,
        pltpu.SemaphoreType.DMA,
    ],
    compiler_params=pltpu.CompilerParams(needs_layout_passes=False,
                                         use_tc_tiling_on_sc=False),
)
def _scatter_kernel(u_hbm, src_hbm, dst_hbm, out_hbm,
                    src_v, dst_v, r0b, r1b, r2b, r3b, acc, s0, s1, s2, s3):
    rows = (r0b, r1b, r2b, r3b)
    sems = (s0, s1, s2, s3)
    cid = lax.axis_index("c")
    sid = lax.axis_index("s")
    u_half = u_hbm.at[pl.ds(cid * N_PAD, N_PAD)]
    pltpu.sync_copy(src_hbm.at[sid], src_v)
    pltpu.sync_copy(dst_hbm.at[sid], dst_v)
    r0 = sid * ROWS_PER_TILE
    pltpu.sync_copy(u_hbm.at[pl.ds(cid * N_PAD + r0, ROWS_PER_TILE)],
                    acc.at[pl.ds(r0, ROWS_PER_TILE)])
    plsc.subcore_barrier()

    for b in range(NBUF):
        pltpu.async_copy(u_half.at[src_v.at[b]], rows[b], sems[b])

    def _group(g, carry):
        for b in range(NBUF):
            j = g * NBUF + b
            pltpu.make_async_copy(u_half.at[src_v.at[j]], rows[b],
                                  sems[b]).wait()
            pltpu.sync_copy(rows[b], acc.at[dst_v.at[j]], add=True)
            pltpu.async_copy(u_half.at[src_v.at[j + NBUF]], rows[b], sems[b])
        return carry

    lax.fori_loop(0, NCHUNK // NBUF - 1, _group, 0)
    for b in range(NBUF):
        j = NCHUNK - NBUF + b
        pltpu.make_async_copy(u_half.at[src_v.at[j]], rows[b], sems[b]).wait()
        pltpu.sync_copy(rows[b], acc.at[dst_v.at[j]], add=True)

    plsc.subcore_barrier()
    pltpu.sync_copy(acc.at[pl.ds(r0, ROWS_PER_TILE)],
                    out_hbm.at[cid, pl.ds(r0, ROWS_PER_TILE)])


_BR = 256
_GRID = N_PAD // _BR


def _degsum_body(dp_ref, o_ref):
    o_ref[...] = jnp.sum(dp_ref[...], axis=0)


def _dinv(d):
    return lax.rsqrt(d + 1.0)


def _u_body(x_ref, w_ref, d_ref, u_ref):
    h = jnp.dot(x_ref[...], w_ref[...], preferred_element_type=jnp.float32)
    u = h * _dinv(d_ref[...])
    u_ref[0] = u[:, :COLS]
    u_ref[1] = u[:, COLS:]


def _ln_relu(pre, g, beta):
    mu = jnp.mean(pre, axis=1, keepdims=True)
    var = jnp.mean((pre - mu) ** 2, axis=1, keepdims=True)
    return jnp.maximum((pre - mu) * lax.rsqrt(var + 1e-5) * g + beta, 0.0)


def _mid_body(p_ref, x0_ref, w2_ref, b1_ref, g1_ref,
              be1_ref, d_ref, x1_ref, u2_ref):
    dinv = _dinv(d_ref[...])
    scat = jnp.concatenate([p_ref[0], p_ref[1]], axis=1)
    pre = scat * dinv + b1_ref[...]
    x1 = _ln_relu(pre, g1_ref[...], be1_ref[...]) + x0_ref[...]
    x1_ref[...] = x1
    u2 = jnp.dot(x1, w2_ref[...], preferred_element_type=jnp.float32) * dinv
    u2_ref[0] = u2[:, :COLS]
    u2_ref[1] = u2[:, COLS:]


def _final_body(p_ref, x1_ref, b2_ref, g2_ref, be2_ref,
                d_ref, o_ref):
    dinv = _dinv(d_ref[...])
    scat = jnp.concatenate([p_ref[0], p_ref[1]], axis=1)
    pre = scat * dinv + b2_ref[...]
    o_ref[...] = _ln_relu(pre, g2_ref[...], be2_ref[...]) + x1_ref[...]


def _row_spec():
    return pl.BlockSpec((_BR, D), lambda i: (i, 0))


def _stk_spec():
    return pl.BlockSpec((NC, _BR, COLS), lambda i: (0, i, 0))


def _full_spec():
    return pl.BlockSpec((D, D), lambda i: (0, 0))


def _vec_spec():
    return pl.BlockSpec((1, D), lambda i: (0, 0))


def _col_spec():
    return pl.BlockSpec((_BR, 1), lambda i: (i, 0))


_f32 = jnp.float32


def _degsum_call(degp):
    return pl.pallas_call(
        _degsum_body,
        in_specs=[pl.BlockSpec((NW, N_PAD // D, D), lambda: (0, 0, 0))],
        out_specs=pl.BlockSpec((N_PAD // D, D), lambda: (0, 0)),
        out_shape=jax.ShapeDtypeStruct((N_PAD // D, D), _f32),
    )(degp)


def _u_call(xp, W, d):
    return pl.pallas_call(
        _u_body,
        grid=(_GRID,),
        in_specs=[_row_spec(), _full_spec(), _col_spec()],
        out_specs=_stk_spec(),
        out_shape=jax.ShapeDtypeStruct((NC, N_PAD, COLS), _f32),
    )(xp, W, d)


def _mid_call(parts, x0, W2, b1, g1, be1, d):
    return pl.pallas_call(
        _mid_body,
        grid=(_GRID,),
        in_specs=[_stk_spec(), _row_spec(),
                  _full_spec(), _vec_spec(), _vec_spec(), _vec_spec(),
                  _col_spec()],
        out_specs=[_row_spec(), _stk_spec()],
        out_shape=[jax.ShapeDtypeStruct((N_PAD, D), _f32),
                   jax.ShapeDtypeStruct((NC, N_PAD, COLS), _f32)],
    )(parts, x0, W2, b1, g1, be1, d)


def _final_call(parts, x1, b2, g2, be2, d):
    return pl.pallas_call(
        _final_body,
        grid=(_GRID,),
        in_specs=[_stk_spec(), _row_spec(),
                  _vec_spec(), _vec_spec(), _vec_spec(), _col_spec()],
        out_specs=_row_spec(),
        out_shape=jax.ShapeDtypeStruct((N_PAD, D), _f32),
    )(parts, x1, b2, g2, be2, d)


def kernel(x, edge_index, W1, b1, g1, beta1, W2, b2, g2, beta2):
    ei = edge_index.astype(jnp.int32)
    pad = E_PAD - N_EDGES
    dummy = N_NODES + jnp.arange(pad, dtype=jnp.int32) % (N_PAD - N_NODES)
    src = jnp.concatenate([ei[0], dummy])
    dst = jnp.concatenate([ei[1], dummy])
    src3 = src.reshape(NS, NCHUNK, CHUNK)
    dst3 = dst.reshape(NS, NCHUNK, CHUNK)
    dst_deg = dst.reshape(NW, NCHUNK_DEG, CHUNK)
    xp = jnp.pad(x, ((0, N_PAD - N_NODES), (0, 0)))

    degp = _deg_kernel(dst_deg)
    d = _degsum_call(degp.reshape(NW, N_PAD // D, D)).reshape(N_PAD, 1)

    b1r = b1.reshape(1, D)
    g1r = g1.reshape(1, D)
    be1r = beta1.reshape(1, D)
    b2r = b2.reshape(1, D)
    g2r = g2.reshape(1, D)
    be2r = beta2.reshape(1, D)

    u1 = _u_call(xp, W1, d)
    parts1 = _scatter_kernel(u1.reshape(NC * N_PAD, COLS), src3, dst3)
    x1, u2 = _mid_call(parts1, xp, W2, b1r, g1r, be1r, d)
    parts2 = _scatter_kernel(u2.reshape(NC * N_PAD, COLS), src3, dst3)
    x2 = _final_call(parts2, x1, b2r, g2r, be2r, d)
    return x2[:N_NODES]


# speedup vs baseline: 1.0421x; 1.0115x over previous
import functools

import jax
import jax.numpy as jnp
from jax import lax
from jax.experimental import pallas as pl
from jax.experimental.pallas import tpu as pltpu
from jax.experimental.pallas import tpu_sc as plsc

N_NODES = 10000
D = 128
N_EDGES = 320000

NC = 2
NS = 16
NW = NC * NS
COLS = D // NC
CHUNK = 128
NBUF = 5
E_PAD = 327680
NCHUNK_DEG = E_PAD // (NW * CHUNK)
NCHUNK = E_PAD // (NS * CHUNK)
N_PAD = 10240
ROWS_PER_TILE = N_PAD // NS

_mesh = plsc.VectorSubcoreMesh(core_axis_name="c", subcore_axis_name="s")


@functools.partial(
    pl.kernel,
    out_type=jax.ShapeDtypeStruct((NW, N_PAD), jnp.float32),
    mesh=_mesh,
    scratch_types=[
        pltpu.VMEM((NCHUNK_DEG, CHUNK), jnp.int32),
        pltpu.VMEM((N_PAD,), jnp.float32),
    ],
    compiler_params=pltpu.CompilerParams(needs_layout_passes=False),
)
def _deg_kernel(dst_hbm, out_hbm, dst_v, deg_v):
    cid = lax.axis_index("c")
    sid = lax.axis_index("s")
    wid = cid * NS + sid
    pltpu.sync_copy(dst_hbm.at[wid], dst_v)

    zeros16 = jnp.zeros((16,), jnp.float32)

    def _zero(r, carry):
        deg_v[pl.ds(r * 16, 16)] = zeros16
        return carry

    lax.fori_loop(0, N_PAD // 16, _zero, 0)

    ones16 = jnp.ones((16,), jnp.float32)

    def _edges(j, carry):
        def _sub(k, c2):
            idx = dst_v[j, pl.ds(k * 16, 16)]
            plsc.addupdate_scatter(deg_v, [idx], ones16)
            return c2
        return lax.fori_loop(0, CHUNK // 16, _sub, carry)

    lax.fori_loop(0, NCHUNK_DEG, _edges, 0)
    pltpu.sync_copy(deg_v, out_hbm.at[wid])


@functools.partial(
    pl.kernel,
    out_type=jax.ShapeDtypeStruct((NC, N_PAD, COLS), jnp.float32),
    mesh=_mesh,
    scratch_types=[
        pltpu.VMEM((NCHUNK, CHUNK), jnp.int32),
        pltpu.VMEM((NCHUNK, CHUNK), jnp.int32),
        pltpu.VMEM((CHUNK, COLS), jnp.float32),
        pltpu.VMEM((CHUNK, COLS), jnp.float32),
        pltpu.VMEM((CHUNK, COLS), jnp.float32),
        pltpu.VMEM((CHUNK, COLS), jnp.float32),
        pltpu.VMEM((CHUNK, COLS), jnp.float32),
        pltpu.VMEM_SHARED((N_PAD, COLS), jnp.float32),
        pltpu.SemaphoreType.DMA,
        pltpu.SemaphoreType.DMA,
        pltpu.SemaphoreType.DMA,
        pltpu.SemaphoreType.DMA,
        pltpu.SemaphoreType.DMA,
    ],
    compiler_params=pltpu.CompilerParams(needs_layout_passes=False,
                                         use_tc_tiling_on_sc=False),
)
def _scatter_kernel(u_hbm, src_hbm, dst_hbm, out_hbm,
                    src_v, dst_v, r0b, r1b, r2b, r3b, r4b, acc,
                    s0, s1, s2, s3, s4):
    rows = (r0b, r1b, r2b, r3b, r4b)
    sems = (s0, s1, s2, s3, s4)
    cid = lax.axis_index("c")
    sid = lax.axis_index("s")
    u_half = u_hbm.at[pl.ds(cid * N_PAD, N_PAD)]
    pltpu.sync_copy(src_hbm.at[sid], src_v)
    pltpu.sync_copy(dst_hbm.at[sid], dst_v)
    r0 = sid * ROWS_PER_TILE
    pltpu.sync_copy(u_hbm.at[pl.ds(cid * N_PAD + r0, ROWS_PER_TILE)],
                    acc.at[pl.ds(r0, ROWS_PER_TILE)])
    plsc.subcore_barrier()

    for b in range(NBUF):
        pltpu.async_copy(u_half.at[src_v.at[b]], rows[b], sems[b])

    def _group(g, carry):
        for b in range(NBUF):
            j = g * NBUF + b
            pltpu.make_async_copy(u_half.at[src_v.at[j]], rows[b],
                                  sems[b]).wait()
            pltpu.sync_copy(rows[b], acc.at[dst_v.at[j]], add=True)
            pltpu.async_copy(u_half.at[src_v.at[j + NBUF]], rows[b], sems[b])
        return carry

    lax.fori_loop(0, NCHUNK // NBUF - 1, _group, 0)
    for b in range(NBUF):
        j = NCHUNK - NBUF + b
        pltpu.make_async_copy(u_half.at[src_v.at[j]], rows[b], sems[b]).wait()
        pltpu.sync_copy(rows[b], acc.at[dst_v.at[j]], add=True)

    plsc.subcore_barrier()
    pltpu.sync_copy(acc.at[pl.ds(r0, ROWS_PER_TILE)],
                    out_hbm.at[cid, pl.ds(r0, ROWS_PER_TILE)])


_BR = 256
_GRID = N_PAD // _BR


def _degsum_body(dp_ref, o_ref):
    o_ref[...] = jnp.sum(dp_ref[...], axis=0)


def _dinv(d):
    return lax.rsqrt(d + 1.0)


def _u_body(x_ref, w_ref, d_ref, u_ref):
    h = jnp.dot(x_ref[...], w_ref[...], preferred_element_type=jnp.float32)
    u = h * _dinv(d_ref[...])
    u_ref[0] = u[:, :COLS]
    u_ref[1] = u[:, COLS:]


def _ln_relu(pre, g, beta):
    mu = jnp.mean(pre, axis=1, keepdims=True)
    var = jnp.mean((pre - mu) ** 2, axis=1, keepdims=True)
    return jnp.maximum((pre - mu) * lax.rsqrt(var + 1e-5) * g + beta, 0.0)


def _mid_body(p_ref, x0_ref, w2_ref, b1_ref, g1_ref,
              be1_ref, d_ref, x1_ref, u2_ref):
    dinv = _dinv(d_ref[...])
    scat = jnp.concatenate([p_ref[0], p_ref[1]], axis=1)
    pre = scat * dinv + b1_ref[...]
    x1 = _ln_relu(pre, g1_ref[...], be1_ref[...]) + x0_ref[...]
    x1_ref[...] = x1
    u2 = jnp.dot(x1, w2_ref[...], preferred_element_type=jnp.float32) * dinv
    u2_ref[0] = u2[:, :COLS]
    u2_ref[1] = u2[:, COLS:]


def _final_body(p_ref, x1_ref, b2_ref, g2_ref, be2_ref,
                d_ref, o_ref):
    dinv = _dinv(d_ref[...])
    scat = jnp.concatenate([p_ref[0], p_ref[1]], axis=1)
    pre = scat * dinv + b2_ref[...]
    o_ref[...] = _ln_relu(pre, g2_ref[...], be2_ref[...]) + x1_ref[...]


def _row_spec():
    return pl.BlockSpec((_BR, D), lambda i: (i, 0))


def _stk_spec():
    return pl.BlockSpec((NC, _BR, COLS), lambda i: (0, i, 0))


def _full_spec():
    return pl.BlockSpec((D, D), lambda i: (0, 0))


def _vec_spec():
    return pl.BlockSpec((1, D), lambda i: (0, 0))


def _col_spec():
    return pl.BlockSpec((_BR, 1), lambda i: (i, 0))


_f32 = jnp.float32


def _degsum_call(degp):
    return pl.pallas_call(
        _degsum_body,
        in_specs=[pl.BlockSpec((NW, N_PAD // D, D), lambda: (0, 0, 0))],
        out_specs=pl.BlockSpec((N_PAD // D, D), lambda: (0, 0)),
        out_shape=jax.ShapeDtypeStruct((N_PAD // D, D), _f32),
    )(degp)


def _u_call(xp, W, d):
    return pl.pallas_call(
        _u_body,
        grid=(_GRID,),
        in_specs=[_row_spec(), _full_spec(), _col_spec()],
        out_specs=_stk_spec(),
        out_shape=jax.ShapeDtypeStruct((NC, N_PAD, COLS), _f32),
    )(xp, W, d)


def _mid_call(parts, x0, W2, b1, g1, be1, d):
    return pl.pallas_call(
        _mid_body,
        grid=(_GRID,),
        in_specs=[_stk_spec(), _row_spec(),
                  _full_spec(), _vec_spec(), _vec_spec(), _vec_spec(),
                  _col_spec()],
        out_specs=[_row_spec(), _stk_spec()],
        out_shape=[jax.ShapeDtypeStruct((N_PAD, D), _f32),
                   jax.ShapeDtypeStruct((NC, N_PAD, COLS), _f32)],
    )(parts, x0, W2, b1, g1, be1, d)


def _final_call(parts, x1, b2, g2, be2, d):
    return pl.pallas_call(
        _final_body,
        grid=(_GRID,),
        in_specs=[_stk_spec(), _row_spec(),
                  _vec_spec(), _vec_spec(), _vec_spec(), _col_spec()],
        out_specs=_row_spec(),
        out_shape=jax.ShapeDtypeStruct((N_NODES, D), _f32),
    )(parts, x1, b2, g2, be2, d)


def kernel(x, edge_index, W1, b1, g1, beta1, W2, b2, g2, beta2):
    ei = edge_index.astype(jnp.int32)
    pad = E_PAD - N_EDGES
    dummy = N_NODES + jnp.arange(pad, dtype=jnp.int32) % (N_PAD - N_NODES)
    src = jnp.concatenate([ei[0], dummy])
    dst = jnp.concatenate([ei[1], dummy])
    src3 = src.reshape(NS, NCHUNK, CHUNK)
    dst3 = dst.reshape(NS, NCHUNK, CHUNK)
    dst_deg = dst.reshape(NW, NCHUNK_DEG, CHUNK)
    xp = jnp.pad(x, ((0, N_PAD - N_NODES), (0, 0)))

    degp = _deg_kernel(dst_deg)
    d = _degsum_call(degp.reshape(NW, N_PAD // D, D)).reshape(N_PAD, 1)

    b1r = b1.reshape(1, D)
    g1r = g1.reshape(1, D)
    be1r = beta1.reshape(1, D)
    b2r = b2.reshape(1, D)
    g2r = g2.reshape(1, D)
    be2r = beta2.reshape(1, D)

    u1 = _u_call(xp, W1, d)
    parts1 = _scatter_kernel(u1.reshape(NC * N_PAD, COLS), src3, dst3)
    x1, u2 = _mid_call(parts1, xp, W2, b1r, g1r, be1r, d)
    parts2 = _scatter_kernel(u2.reshape(NC * N_PAD, COLS), src3, dst3)
    return _final_call(parts2, x1, b2r, g2r, be2r, d)
